# baseline (device time: 62632 ns/iter reference)
import jax
import jax.numpy as jnp
from jax import lax
from jax.experimental import pallas as pl
from jax.experimental.pallas import tpu as pltpu

N_DEV = 8


def kernel(x, w_mat):
    m_per, k = x.shape
    _, n = w_mat.shape
    n_per = n // N_DEV

    def body(x_ref, w_ref, out_ref, y_buf, send_sems, recv_sems):
        my = lax.axis_index("i")
        xb = x_ref[:, :].astype(jnp.bfloat16)

        def block(tgt):
            wb = w_ref[:, pl.ds(tgt * n_per, n_per)].astype(jnp.bfloat16)
            y = jnp.dot(xb, wb, preferred_element_type=jnp.float32)
            return y * jax.nn.sigmoid(y)

        out_ref[pl.ds(my * m_per, m_per), :] = block(my)

        rdmas = []
        for s in range(1, N_DEV):
            tgt = lax.rem(my + s, N_DEV)
            y_buf[s - 1, :, :] = block(tgt)
            rdma = pltpu.make_async_remote_copy(
                src_ref=y_buf.at[s - 1],
                dst_ref=out_ref.at[pl.ds(my * m_per, m_per), :],
                send_sem=send_sems.at[s - 1],
                recv_sem=recv_sems.at[s - 1],
                device_id=(tgt,),
                device_id_type=pl.DeviceIdType.MESH,
            )
            rdma.start()
            rdmas.append(rdma)

        for s in range(1, N_DEV):
            src = lax.rem(my - s + N_DEV, N_DEV)
            rdmas[s - 1].wait_send()
            recv = pltpu.make_async_remote_copy(
                src_ref=y_buf.at[s - 1],
                dst_ref=out_ref.at[pl.ds(src * m_per, m_per), :],
                send_sem=send_sems.at[s - 1],
                recv_sem=recv_sems.at[s - 1],
                device_id=(src,),
                device_id_type=pl.DeviceIdType.MESH,
            )
            recv.wait_recv()

    return pl.pallas_call(
        body,
        out_shape=jax.ShapeDtypeStruct((N_DEV * m_per, n_per), jnp.float32),
        in_specs=[
            pl.BlockSpec(memory_space=pltpu.VMEM),
            pl.BlockSpec(memory_space=pltpu.VMEM),
        ],
        out_specs=pl.BlockSpec(memory_space=pltpu.VMEM),
        scratch_shapes=[
            pltpu.VMEM((N_DEV - 1, m_per, n_per), jnp.float32),
            pltpu.SemaphoreType.DMA((N_DEV - 1,)),
            pltpu.SemaphoreType.DMA((N_DEV - 1,)),
        ],
        compiler_params=pltpu.CompilerParams(
            vmem_limit_bytes=100 * 1024 * 1024,
        ),
    )(x, w_mat)


# device time: 49007 ns/iter; 1.2780x vs baseline; 1.2780x over previous
import jax
import jax.numpy as jnp
from jax import lax
from jax.experimental import pallas as pl
from jax.experimental.pallas import tpu as pltpu

N_DEV = 8


def kernel(x, w_mat):
    m_per, k = x.shape
    _, n = w_mat.shape
    n_per = n // N_DEV

    def body(x_ref, w_ref, out_ref, send_buf, recv_buf, send_sems, recv_sems):
        my = lax.axis_index("i")
        xb = x_ref[:, :].astype(jnp.bfloat16)

        def block(tgt):
            wb = w_ref[:, pl.ds(tgt * n_per, n_per)].astype(jnp.bfloat16)
            y = jnp.dot(xb, wb, preferred_element_type=jnp.float32)
            return y * jax.nn.sigmoid(y)

        out_ref[pl.ds(my * m_per, m_per), :] = block(my)

        rdmas = []
        for s in range(1, N_DEV):
            tgt = lax.rem(my + s, N_DEV)
            send_buf[s - 1, :, :] = block(tgt).astype(jnp.bfloat16)
            rdma = pltpu.make_async_remote_copy(
                src_ref=send_buf.at[s - 1],
                dst_ref=recv_buf.at[s - 1],
                send_sem=send_sems.at[s - 1],
                recv_sem=recv_sems.at[s - 1],
                device_id=(tgt,),
                device_id_type=pl.DeviceIdType.MESH,
            )
            rdma.start()
            rdmas.append(rdma)

        for s in range(1, N_DEV):
            src = lax.rem(my - s + N_DEV, N_DEV)
            rdmas[s - 1].wait_send()
            rdmas[s - 1].wait_recv()
            out_ref[pl.ds(src * m_per, m_per), :] = recv_buf[s - 1, :, :].astype(
                jnp.float32
            )

    return pl.pallas_call(
        body,
        out_shape=jax.ShapeDtypeStruct((N_DEV * m_per, n_per), jnp.float32),
        in_specs=[
            pl.BlockSpec(memory_space=pltpu.VMEM),
            pl.BlockSpec(memory_space=pltpu.VMEM),
        ],
        out_specs=pl.BlockSpec(memory_space=pltpu.VMEM),
        scratch_shapes=[
            pltpu.VMEM((N_DEV - 1, m_per, n_per), jnp.bfloat16),
            pltpu.VMEM((N_DEV - 1, m_per, n_per), jnp.bfloat16),
            pltpu.SemaphoreType.DMA((N_DEV - 1,)),
            pltpu.SemaphoreType.DMA((N_DEV - 1,)),
        ],
        compiler_params=pltpu.CompilerParams(
            vmem_limit_bytes=100 * 1024 * 1024,
        ),
    )(x, w_mat)


# device time: 46387 ns/iter; 1.3502x vs baseline; 1.0565x over previous
import jax
import jax.numpy as jnp
from jax import lax
from jax.experimental import pallas as pl
from jax.experimental.pallas import tpu as pltpu

N_DEV = 8


def kernel(x, w_mat):
    m_per, k = x.shape
    _, n = w_mat.shape
    n_per = n // N_DEV

    def body(x_ref, w_ref, out_ref, send_buf, recv_buf, send_sems, recv_sems):
        my = lax.axis_index("i")

        barrier_sem = pltpu.get_barrier_semaphore()
        for s in range(1, N_DEV):
            pl.semaphore_signal(
                barrier_sem, inc=1,
                device_id=(lax.rem(my + s, N_DEV),),
                device_id_type=pl.DeviceIdType.MESH,
            )
        pl.semaphore_wait(barrier_sem, N_DEV - 1)

        xb = x_ref[:, :].astype(jnp.bfloat16)

        def block(tgt):
            wb = w_ref[:, pl.ds(tgt * n_per, n_per)].astype(jnp.bfloat16)
            y = jnp.dot(xb, wb, preferred_element_type=jnp.float32)
            return y * jax.nn.sigmoid(y)

        out_ref[pl.ds(my * m_per, m_per), :] = block(my)

        rdmas = []
        for s in range(1, N_DEV):
            tgt = lax.rem(my + s, N_DEV)
            send_buf[s - 1, :, :] = block(tgt).astype(jnp.bfloat16)
            rdma = pltpu.make_async_remote_copy(
                src_ref=send_buf.at[s - 1],
                dst_ref=recv_buf.at[s - 1],
                send_sem=send_sems.at[s - 1],
                recv_sem=recv_sems.at[s - 1],
                device_id=(tgt,),
                device_id_type=pl.DeviceIdType.MESH,
            )
            rdma.start()
            rdmas.append(rdma)

        for s in range(1, N_DEV):
            src = lax.rem(my - s + N_DEV, N_DEV)
            rdmas[s - 1].wait_send()
            rdmas[s - 1].wait_recv()
            out_ref[pl.ds(src * m_per, m_per), :] = recv_buf[s - 1, :, :].astype(
                jnp.float32
            )

    return pl.pallas_call(
        body,
        out_shape=jax.ShapeDtypeStruct((N_DEV * m_per, n_per), jnp.float32),
        in_specs=[
            pl.BlockSpec(memory_space=pltpu.VMEM),
            pl.BlockSpec(memory_space=pltpu.VMEM),
        ],
        out_specs=pl.BlockSpec(memory_space=pltpu.VMEM),
        scratch_shapes=[
            pltpu.VMEM((N_DEV - 1, m_per, n_per), jnp.bfloat16),
            pltpu.VMEM((N_DEV - 1, m_per, n_per), jnp.bfloat16),
            pltpu.SemaphoreType.DMA((N_DEV - 1,)),
            pltpu.SemaphoreType.DMA((N_DEV - 1,)),
        ],
        compiler_params=pltpu.CompilerParams(
            vmem_limit_bytes=100 * 1024 * 1024,
            collective_id=0,
        ),
    )(x, w_mat)
